# baseline (speedup 1.0000x reference)
import jax
import jax.numpy as jnp
from jax import lax
from jax.experimental import pallas as pl
from jax.experimental.pallas import tpu as pltpu
from jax.experimental.pallas import tpu_sc as plsc

_NC = 2
_NS = 16
_NW = _NC * _NS
_CR = 8
_BM = 1024
_TC_ROWS = 16384


def _sc_body(x_hbm, o_hbm, in_a, in_b, out_a, out_b, s_la, s_lb, s_sa, s_sb):
    m, n = x_hbm.shape
    sc_rows = m - _TC_ROWS
    rows_w = sc_rows // _NW
    nchunks = rows_w // _CR

    wid = lax.axis_index("s") * _NC + lax.axis_index("c")
    base = _TC_ROWS + wid * rows_w
    obase = wid * rows_w

    iota = lax.broadcasted_iota(jnp.int32, (16,), 0)
    swap = iota ^ 1
    even = (iota & 1) == 0

    def compute(ibuf, obuf):
        for r in range(_CR):
            @plsc.parallel_loop(0, n // 16, unroll=8)
            def _(i):
                off = i * 16
                v = ibuf[r, pl.ds(off, 16)]
                p = lax.gather(
                    v, swap[:, None],
                    lax.GatherDimensionNumbers(
                        offset_dims=(), collapsed_slice_dims=(0,),
                        start_index_map=(0,)),
                    (1,),
                    unique_indices=True,
                    mode=lax.GatherScatterMode.PROMISE_IN_BOUNDS)
                obuf[r, pl.ds(off, 16)] = jnp.where(
                    even, jnp.minimum(v, p), jnp.maximum(v, p))

    def load(g, buf, sem):
        pltpu.make_async_copy(
            x_hbm.at[pl.ds(base + g * _CR, _CR), :], buf, sem).start()

    def load_wait(g, buf, sem):
        pltpu.make_async_copy(
            x_hbm.at[pl.ds(base + g * _CR, _CR), :], buf, sem).wait()

    def store(g, buf, sem):
        pltpu.make_async_copy(
            buf, o_hbm.at[pl.ds(obase + g * _CR, _CR), :], sem).start()

    def store_wait(g, buf, sem):
        pltpu.make_async_copy(
            buf, o_hbm.at[pl.ds(obase + g * _CR, _CR), :], sem).wait()

    load(0, in_a, s_la)

    @pl.loop(0, nchunks, step=2)
    def _(g):
        load(g + 1, in_b, s_lb)
        load_wait(g, in_a, s_la)

        @pl.when(g > 0)
        def _():
            store_wait(g - 2, out_a, s_sa)

        compute(in_a, out_a)
        store(g, out_a, s_sa)

        @pl.when(g + 2 < nchunks)
        def _():
            load(g + 2, in_a, s_la)

        load_wait(g + 1, in_b, s_lb)

        @pl.when(g > 0)
        def _():
            store_wait(g - 1, out_b, s_sb)

        compute(in_b, out_b)
        store(g + 1, out_b, s_sb)

    store_wait(nchunks - 2, out_a, s_sa)
    store_wait(nchunks - 1, out_b, s_sb)


def _sc_part(input):
    m, n = input.shape
    return pl.kernel(
        _sc_body,
        out_type=jax.ShapeDtypeStruct((m - _TC_ROWS, n), input.dtype),
        mesh=plsc.VectorSubcoreMesh(core_axis_name="c", subcore_axis_name="s"),
        scratch_types=[
            pltpu.VMEM((_CR, n), jnp.float32),
            pltpu.VMEM((_CR, n), jnp.float32),
            pltpu.VMEM((_CR, n), jnp.float32),
            pltpu.VMEM((_CR, n), jnp.float32),
            pltpu.SemaphoreType.DMA,
            pltpu.SemaphoreType.DMA,
            pltpu.SemaphoreType.DMA,
            pltpu.SemaphoreType.DMA,
        ],
    )(input)


def _groupsort2_block(x_ref, o_ref):
    x = x_ref[...]
    bm, n = x.shape
    parity_even = (lax.broadcasted_iota(jnp.int32, (bm, n), 1) & 1) == 0
    left = jnp.roll(x, -1, axis=1)
    right = jnp.roll(x, 1, axis=1)
    partner = jnp.where(parity_even, left, right)
    o_ref[...] = jnp.where(parity_even,
                           jnp.minimum(x, partner),
                           jnp.maximum(x, partner))


def _tc_part(input):
    m, n = input.shape
    grid = (_TC_ROWS // _BM,)
    return pl.pallas_call(
        _groupsort2_block,
        grid=grid,
        in_specs=[pl.BlockSpec((_BM, n), lambda i: (i, 0))],
        out_specs=pl.BlockSpec((_BM, n), lambda i: (i, 0)),
        out_shape=jax.ShapeDtypeStruct((_TC_ROWS, n), input.dtype),
    )(input)


def kernel(input):
    sc_out = _sc_part(input)
    tc_out = _tc_part(input)
    return jnp.concatenate([tc_out, sc_out], axis=0)


# speedup vs baseline: 1.6927x; 1.6927x over previous
import jax
import jax.numpy as jnp
from jax import lax
from jax.experimental import pallas as pl
from jax.experimental.pallas import tpu as pltpu
from jax.experimental.pallas import tpu_sc as plsc

_NC = 2
_NS = 16
_NW = _NC * _NS
_CR = 16


def _sc_body(x_hbm, o_hbm, buf_a, buf_b, buf_c, s_a, s_b, s_c,
             t_a, t_b, t_c):
    m, n = x_hbm.shape
    rows_w = m // _NW
    nchunks = rows_w // _CR

    wid = lax.axis_index("s") * _NC + lax.axis_index("c")
    base = wid * rows_w

    iota = lax.broadcasted_iota(jnp.int32, (16,), 0)
    swap = iota ^ 1
    even = (iota & 1) == 0

    def compute(buf):
        for r in range(_CR):
            @plsc.parallel_loop(0, n // 16, unroll=8)
            def _(i):
                off = i * 16
                v = buf[r, pl.ds(off, 16)]
                p = lax.gather(
                    v, swap[:, None],
                    lax.GatherDimensionNumbers(
                        offset_dims=(), collapsed_slice_dims=(0,),
                        start_index_map=(0,)),
                    (1,),
                    unique_indices=True,
                    mode=lax.GatherScatterMode.PROMISE_IN_BOUNDS)
                buf[r, pl.ds(off, 16)] = jnp.where(
                    even, jnp.minimum(v, p), jnp.maximum(v, p))

    def load(g, buf, sem):
        pltpu.make_async_copy(
            x_hbm.at[pl.ds(base + g * _CR, _CR), :], buf, sem).start()

    def load_wait(g, buf, sem):
        pltpu.make_async_copy(
            x_hbm.at[pl.ds(base + g * _CR, _CR), :], buf, sem).wait()

    def store(g, buf, sem):
        pltpu.make_async_copy(
            buf, o_hbm.at[pl.ds(base + g * _CR, _CR), :], sem).start()

    def store_wait(g, buf, sem):
        pltpu.make_async_copy(
            buf, o_hbm.at[pl.ds(base + g * _CR, _CR), :], sem).wait()

    load(0, buf_a, s_a)
    load(1, buf_b, s_b)

    @pl.loop(0, nchunks - 1, step=3)
    def _(g):
        load_wait(g, buf_a, s_a)
        compute(buf_a)
        store(g, buf_a, t_a)

        @pl.when(g > 0)
        def _():
            store_wait(g - 1, buf_c, t_c)

        load(g + 2, buf_c, s_c)

        load_wait(g + 1, buf_b, s_b)
        compute(buf_b)
        store(g + 1, buf_b, t_b)
        store_wait(g, buf_a, t_a)
        load(g + 3, buf_a, s_a)

        load_wait(g + 2, buf_c, s_c)
        compute(buf_c)
        store(g + 2, buf_c, t_c)

        @pl.when(g + 4 < nchunks)
        def _():
            store_wait(g + 1, buf_b, t_b)
            load(g + 4, buf_b, s_b)

    load_wait(nchunks - 1, buf_a, s_a)
    compute(buf_a)
    store(nchunks - 1, buf_a, t_a)
    store_wait(nchunks - 2, buf_c, t_c)
    store_wait(nchunks - 3, buf_b, t_b)
    store_wait(nchunks - 1, buf_a, t_a)


def kernel(input):
    m, n = input.shape
    return pl.kernel(
        _sc_body,
        out_type=jax.ShapeDtypeStruct((m, n), input.dtype),
        mesh=plsc.VectorSubcoreMesh(core_axis_name="c", subcore_axis_name="s"),
        scratch_types=[
            pltpu.VMEM((_CR, n), jnp.float32),
            pltpu.VMEM((_CR, n), jnp.float32),
            pltpu.VMEM((_CR, n), jnp.float32),
            pltpu.SemaphoreType.DMA,
            pltpu.SemaphoreType.DMA,
            pltpu.SemaphoreType.DMA,
            pltpu.SemaphoreType.DMA,
            pltpu.SemaphoreType.DMA,
            pltpu.SemaphoreType.DMA,
        ],
    )(input)


# speedup vs baseline: 1.7035x; 1.0063x over previous
import jax
import jax.numpy as jnp
from jax import lax
from jax.experimental import pallas as pl
from jax.experimental.pallas import tpu as pltpu
from jax.experimental.pallas import tpu_sc as plsc

_NC = 2
_NS = 16
_NW = _NC * _NS
_CR = 8


def _sc_body(x_hbm, o_hbm, in_a, in_b, out_a, out_b, s_la, s_lb, s_sa, s_sb):
    m, n = x_hbm.shape
    rows_w = m // _NW
    nchunks = rows_w // _CR

    wid = lax.axis_index("s") * _NC + lax.axis_index("c")
    base = wid * rows_w

    iota = lax.broadcasted_iota(jnp.int32, (16,), 0)
    swap = iota ^ 1
    even = (iota & 1) == 0

    def compute(ibuf, obuf):
        for r in range(_CR):
            @plsc.parallel_loop(0, n // 16, unroll=8)
            def _(i):
                off = i * 16
                v = ibuf[r, pl.ds(off, 16)]
                p = lax.gather(
                    v, swap[:, None],
                    lax.GatherDimensionNumbers(
                        offset_dims=(), collapsed_slice_dims=(0,),
                        start_index_map=(0,)),
                    (1,),
                    unique_indices=True,
                    mode=lax.GatherScatterMode.PROMISE_IN_BOUNDS)
                obuf[r, pl.ds(off, 16)] = jnp.where(
                    even, jnp.minimum(v, p), jnp.maximum(v, p))

    def load(g, buf, sem):
        pltpu.make_async_copy(
            x_hbm.at[pl.ds(base + g * _CR, _CR), :], buf, sem).start()

    def load_wait(g, buf, sem):
        pltpu.make_async_copy(
            x_hbm.at[pl.ds(base + g * _CR, _CR), :], buf, sem).wait()

    def store(g, buf, sem):
        pltpu.make_async_copy(
            buf, o_hbm.at[pl.ds(base + g * _CR, _CR), :], sem).start()

    def store_wait(g, buf, sem):
        pltpu.make_async_copy(
            buf, o_hbm.at[pl.ds(base + g * _CR, _CR), :], sem).wait()

    load(0, in_a, s_la)

    @pl.loop(0, nchunks, step=2)
    def _(g):
        load(g + 1, in_b, s_lb)
        load_wait(g, in_a, s_la)

        @pl.when(g > 0)
        def _():
            store_wait(g - 2, out_a, s_sa)

        compute(in_a, out_a)
        store(g, out_a, s_sa)

        @pl.when(g + 2 < nchunks)
        def _():
            load(g + 2, in_a, s_la)

        load_wait(g + 1, in_b, s_lb)

        @pl.when(g > 0)
        def _():
            store_wait(g - 1, out_b, s_sb)

        compute(in_b, out_b)
        store(g + 1, out_b, s_sb)

    store_wait(nchunks - 2, out_a, s_sa)
    store_wait(nchunks - 1, out_b, s_sb)


def kernel(input):
    m, n = input.shape
    return pl.kernel(
        _sc_body,
        out_type=jax.ShapeDtypeStruct((m, n), input.dtype),
        mesh=plsc.VectorSubcoreMesh(core_axis_name="c", subcore_axis_name="s"),
        scratch_types=[
            pltpu.VMEM((_CR, n), jnp.float32),
            pltpu.VMEM((_CR, n), jnp.float32),
            pltpu.VMEM((_CR, n), jnp.float32),
            pltpu.VMEM((_CR, n), jnp.float32),
            pltpu.SemaphoreType.DMA,
            pltpu.SemaphoreType.DMA,
            pltpu.SemaphoreType.DMA,
            pltpu.SemaphoreType.DMA,
        ],
    )(input)


# speedup vs baseline: 1.7250x; 1.0126x over previous
import jax
import jax.numpy as jnp
from jax import lax
from jax.experimental import pallas as pl
from jax.experimental.pallas import tpu as pltpu
from jax.experimental.pallas import tpu_sc as plsc

_NC = 2
_NS = 16
_NW = _NC * _NS
_CR = 8


def _sc_body(x_hbm, o_hbm, in0, in1, in2, out0, out1, out2,
             l0, l1, l2, s0, s1, s2):
    m, n = x_hbm.shape
    rows_w = m // _NW
    nchunks = rows_w // _CR

    wid = lax.axis_index("s") * _NC + lax.axis_index("c")
    base = wid * rows_w

    iota = lax.broadcasted_iota(jnp.int32, (16,), 0)
    swap = iota ^ 1
    even = (iota & 1) == 0

    ins = (in0, in1, in2)
    outs = (out0, out1, out2)
    lsems = (l0, l1, l2)
    ssems = (s0, s1, s2)

    def compute(ibuf, obuf):
        for r in range(_CR):
            @plsc.parallel_loop(0, n // 16, unroll=8)
            def _(i):
                off = i * 16
                v = ibuf[r, pl.ds(off, 16)]
                p = lax.gather(
                    v, swap[:, None],
                    lax.GatherDimensionNumbers(
                        offset_dims=(), collapsed_slice_dims=(0,),
                        start_index_map=(0,)),
                    (1,),
                    unique_indices=True,
                    mode=lax.GatherScatterMode.PROMISE_IN_BOUNDS)
                obuf[r, pl.ds(off, 16)] = jnp.where(
                    even, jnp.minimum(v, p), jnp.maximum(v, p))

    def load(g, buf, sem):
        pltpu.make_async_copy(
            x_hbm.at[pl.ds(base + g * _CR, _CR), :], buf, sem).start()

    def load_wait(g, buf, sem):
        pltpu.make_async_copy(
            x_hbm.at[pl.ds(base + g * _CR, _CR), :], buf, sem).wait()

    def store(g, buf, sem):
        pltpu.make_async_copy(
            buf, o_hbm.at[pl.ds(base + g * _CR, _CR), :], sem).start()

    def store_wait(g, buf, sem):
        pltpu.make_async_copy(
            buf, o_hbm.at[pl.ds(base + g * _CR, _CR), :], sem).wait()

    load(0, in0, l0)
    load(1, in1, l1)

    @pl.loop(0, nchunks - 2, step=3)
    def _(g):
        for k in range(3):
            c = g + k
            kn = (k + 2) % 3
            load(c + 2, ins[kn], lsems[kn])
            load_wait(c, ins[k], lsems[k])

            @pl.when(c > 2)
            def _():
                store_wait(c - 3, outs[k], ssems[k])

            compute(ins[k], outs[k])
            store(c, outs[k], ssems[k])

    for c, k in ((nchunks - 2, 0), (nchunks - 1, 1)):
        load_wait(c, ins[k], lsems[k])
        store_wait(c - 3, outs[k], ssems[k])
        compute(ins[k], outs[k])
        store(c, outs[k], ssems[k])

    store_wait(nchunks - 3, out2, s2)
    store_wait(nchunks - 2, out0, s0)
    store_wait(nchunks - 1, out1, s1)


def kernel(input):
    m, n = input.shape
    return pl.kernel(
        _sc_body,
        out_type=jax.ShapeDtypeStruct((m, n), input.dtype),
        mesh=plsc.VectorSubcoreMesh(core_axis_name="c", subcore_axis_name="s"),
        scratch_types=[
            pltpu.VMEM((_CR, n), jnp.float32),
            pltpu.VMEM((_CR, n), jnp.float32),
            pltpu.VMEM((_CR, n), jnp.float32),
            pltpu.VMEM((_CR, n), jnp.float32),
            pltpu.VMEM((_CR, n), jnp.float32),
            pltpu.VMEM((_CR, n), jnp.float32),
            pltpu.SemaphoreType.DMA,
            pltpu.SemaphoreType.DMA,
            pltpu.SemaphoreType.DMA,
            pltpu.SemaphoreType.DMA,
            pltpu.SemaphoreType.DMA,
            pltpu.SemaphoreType.DMA,
        ],
    )(input)


# speedup vs baseline: 1.7275x; 1.0015x over previous
import jax
import jax.numpy as jnp
from jax import lax
from jax.experimental import pallas as pl
from jax.experimental.pallas import tpu as pltpu
from jax.experimental.pallas import tpu_sc as plsc

_NC = 2
_NS = 16
_NW = _NC * _NS
_CR = 8


def _sc_body(x_hbm, o_hbm, in0, in1, in2, in3, out0, out1,
             l0, l1, l2, l3, s0, s1):
    m, n = x_hbm.shape
    rows_w = m // _NW
    nchunks = rows_w // _CR

    wid = lax.axis_index("s") * _NC + lax.axis_index("c")
    base = wid * rows_w

    iota = lax.broadcasted_iota(jnp.int32, (16,), 0)
    swap = iota ^ 1
    even = (iota & 1) == 0

    ins = (in0, in1, in2, in3)
    outs = (out0, out1)
    lsems = (l0, l1, l2, l3)
    ssems = (s0, s1)

    def compute(ibuf, obuf):
        for r in range(_CR):
            @plsc.parallel_loop(0, n // 16, unroll=8)
            def _(i):
                off = i * 16
                v = ibuf[r, pl.ds(off, 16)]
                p = lax.gather(
                    v, swap[:, None],
                    lax.GatherDimensionNumbers(
                        offset_dims=(), collapsed_slice_dims=(0,),
                        start_index_map=(0,)),
                    (1,),
                    unique_indices=True,
                    mode=lax.GatherScatterMode.PROMISE_IN_BOUNDS)
                obuf[r, pl.ds(off, 16)] = jnp.where(
                    even, jnp.minimum(v, p), jnp.maximum(v, p))

    def load(g, buf, sem):
        pltpu.make_async_copy(
            x_hbm.at[pl.ds(base + g * _CR, _CR), :], buf, sem).start()

    def load_wait(g, buf, sem):
        pltpu.make_async_copy(
            x_hbm.at[pl.ds(base + g * _CR, _CR), :], buf, sem).wait()

    def store(g, buf, sem):
        pltpu.make_async_copy(
            buf, o_hbm.at[pl.ds(base + g * _CR, _CR), :], sem).start()

    def store_wait(g, buf, sem):
        pltpu.make_async_copy(
            buf, o_hbm.at[pl.ds(base + g * _CR, _CR), :], sem).wait()

    load(0, in0, l0)
    load(1, in1, l1)
    load(2, in2, l2)

    @pl.loop(0, nchunks, step=4)
    def _(g):
        for k in range(4):
            c = g + k
            kn = (k + 3) % 4
            ko = k % 2

            @pl.when(c + 3 < nchunks)
            def _():
                load(c + 3, ins[kn], lsems[kn])

            load_wait(c, ins[k], lsems[k])

            @pl.when(c > 1)
            def _():
                store_wait(c - 2, outs[ko], ssems[ko])

            compute(ins[k], outs[ko])
            store(c, outs[ko], ssems[ko])

    store_wait(nchunks - 2, out0, s0)
    store_wait(nchunks - 1, out1, s1)


def kernel(input):
    m, n = input.shape
    return pl.kernel(
        _sc_body,
        out_type=jax.ShapeDtypeStruct((m, n), input.dtype),
        mesh=plsc.VectorSubcoreMesh(core_axis_name="c", subcore_axis_name="s"),
        scratch_types=[
            pltpu.VMEM((_CR, n), jnp.float32),
            pltpu.VMEM((_CR, n), jnp.float32),
            pltpu.VMEM((_CR, n), jnp.float32),
            pltpu.VMEM((_CR, n), jnp.float32),
            pltpu.VMEM((_CR, n), jnp.float32),
            pltpu.VMEM((_CR, n), jnp.float32),
            pltpu.SemaphoreType.DMA,
            pltpu.SemaphoreType.DMA,
            pltpu.SemaphoreType.DMA,
            pltpu.SemaphoreType.DMA,
            pltpu.SemaphoreType.DMA,
            pltpu.SemaphoreType.DMA,
        ],
    )(input)
